# baseline (speedup 1.0000x reference)
import functools

import jax
import jax.numpy as jnp
from jax.experimental import pallas as pl
from jax.experimental.pallas import tpu as pltpu

K = 64


def _enc_kernel(x_ref, w_ref, b_ref, post_ref, gm_ref):
    pre = jax.lax.dot_general(
        x_ref[...], w_ref[...],
        dimension_numbers=(((1,), (1,)), ((), ())),
        preferred_element_type=jnp.float32,
    )
    post = jnp.maximum(pre + b_ref[...][None, :], 0.0)
    post_ref[...] = post

    bits = jax.lax.bitcast_convert_type(post, jnp.int32)
    pm = bits[:, 0:128]
    for j in range(1, bits.shape[1] // 128):
        pm = jnp.maximum(pm, bits[:, j * 128:(j + 1) * 128])

    @pl.when(pl.program_id(0) == 0)
    def _():
        gm_ref[...] = pm

    @pl.when(pl.program_id(0) != 0)
    def _():
        gm_ref[...] = jnp.maximum(gm_ref[...], pm)


def _rowcount_ge(bits, mid, splits=8):
    n = bits.shape[1]
    w = n // splits
    tot = None
    for s in range(splits):
        c = jnp.sum((bits[:, s * w:(s + 1) * w] >= mid).astype(jnp.int32),
                    axis=1, keepdims=True)
        tot = c if tot is None else tot + c
    return tot


def _select_kernel(post_ref, gm_ref, t_ref,
                   lo_ref, hi_ref, clo_ref, chi_ref, tb_ref, *, k):
    post = post_ref[...]
    rows = post.shape[0]
    bits = jax.lax.bitcast_convert_type(post, jnp.int32)
    cm = gm_ref[...]

    def lb_body(i, p):
        cand = jnp.bitwise_or(p, jnp.int32(1) << (30 - i))
        cnt = jnp.sum((cm >= cand).astype(jnp.int32), axis=1, keepdims=True)
        return jnp.where(cnt >= k, cand, p)

    lo0 = jax.lax.fori_loop(0, 31, lb_body, jnp.zeros((rows, 1), jnp.int32))
    hi0 = jnp.max(cm, axis=1, keepdims=True) + 1

    cnt0 = _rowcount_ge(bits, lo0)
    lo_ref[...] = lo0
    hi_ref[...] = hi0
    clo_ref[...] = cnt0
    chi_ref[...] = jnp.zeros((rows, 1), jnp.int32)
    tb_ref[...] = jnp.where(cnt0 == k, lo0, jnp.full((rows, 1), -1, jnp.int32))

    def cond(state):
        it, n_act = state
        return jnp.logical_and(it < 60, n_act > 0)

    def step(it_s, use_interp):
        lo = lo_ref[...]
        hi = hi_ref[...]
        clo = clo_ref[...]
        chi = chi_ref[...]
        tb = tb_ref[...]
        active = (hi - lo > 1) & (tb < 0)

        bisect = lo + ((hi - lo) >> 1)
        if use_interp:
            lo_f = jax.lax.bitcast_convert_type(lo, jnp.float32)
            hi_f = jax.lax.bitcast_convert_type(hi, jnp.float32)
            frac = (clo - k).astype(jnp.float32) / jnp.maximum(
                (clo - chi).astype(jnp.float32), 1.0)
            interp = jax.lax.bitcast_convert_type(
                lo_f + frac * (hi_f - lo_f), jnp.int32)
            mid = jnp.where(it_s < 36, interp, bisect)
        else:
            mid = bisect
        mid = jnp.clip(mid, lo + 1, hi - 1)

        cnt = _rowcount_ge(bits, mid)
        hit = (cnt == k) & active
        ge = cnt >= k
        tb_ref[...] = jnp.where(hit, mid, tb)
        adv = active & ge & (~hit)
        lo_ref[...] = jnp.where(adv, mid, lo)
        clo_ref[...] = jnp.where(adv, cnt, clo)
        shr = active & (~ge)
        hi_ref[...] = jnp.where(shr, mid, hi)
        chi_ref[...] = jnp.where(shr, cnt, chi)

    def body(state):
        it, _ = state
        step(it, True)
        step(it + 1, True)
        step(it + 2, False)
        n_act = jnp.sum(((hi_ref[...] - lo_ref[...] > 1)
                         & (tb_ref[...] < 0)).astype(jnp.int32))
        return it + 3, n_act

    n0 = jnp.sum(((hi0 - lo0 > 1) & (cnt0 != k)).astype(jnp.int32))
    jax.lax.while_loop(cond, body, (jnp.int32(0), n0))
    thresh = jnp.where(tb_ref[...] < 0, lo_ref[...], tb_ref[...])
    t_ref[...] = jnp.broadcast_to(
        jax.lax.bitcast_convert_type(thresh, jnp.float32), t_ref.shape)


def _dec_kernel(post_ref, w_ref, b_ref, t_ref, xhat_ref, f_ref):
    post = post_ref[...]
    fblk = jnp.where(post >= t_ref[:, 0:1], post, 0.0)
    f_ref[...] = fblk

    @pl.when(pl.program_id(0) == 0)
    def _():
        xhat_ref[...] = jnp.broadcast_to(b_ref[...][None, :], xhat_ref.shape)

    xhat_ref[...] += jax.lax.dot_general(
        fblk, w_ref[...],
        dimension_numbers=(((1,), (1,)), ((), ())),
        preferred_element_type=jnp.float32,
    )


def kernel(x, W_enc, b_enc, W_dec, b_dec):
    batch, d_model = x.shape
    n_features = W_enc.shape[0]

    bf = min(1024, n_features)
    nfb = n_features // bf

    post, gm = pl.pallas_call(
        _enc_kernel,
        grid=(nfb,),
        in_specs=[
            pl.BlockSpec((batch, d_model), lambda i: (0, 0)),
            pl.BlockSpec((bf, d_model), lambda i: (i, 0)),
            pl.BlockSpec((bf,), lambda i: (i,)),
        ],
        out_specs=[
            pl.BlockSpec((batch, bf), lambda i: (0, i)),
            pl.BlockSpec((batch, 128), lambda i: (0, 0)),
        ],
        out_shape=[
            jax.ShapeDtypeStruct((batch, n_features), jnp.float32),
            jax.ShapeDtypeStruct((batch, 128), jnp.int32),
        ],
        compiler_params=pltpu.CompilerParams(
            dimension_semantics=("arbitrary",),
        ),
    )(x, W_enc, b_enc)

    rb = min(64, batch)
    tf = pl.pallas_call(
        functools.partial(_select_kernel, k=K),
        grid=(batch // rb,),
        in_specs=[
            pl.BlockSpec((rb, n_features), lambda i: (i, 0)),
            pl.BlockSpec((rb, 128), lambda i: (i, 0)),
        ],
        out_specs=pl.BlockSpec((rb, 128), lambda i: (i, 0)),
        out_shape=jax.ShapeDtypeStruct((batch, 128), jnp.float32),
        scratch_shapes=[
            pltpu.VMEM((rb, 1), jnp.int32),
            pltpu.VMEM((rb, 1), jnp.int32),
            pltpu.VMEM((rb, 1), jnp.int32),
            pltpu.VMEM((rb, 1), jnp.int32),
            pltpu.VMEM((rb, 1), jnp.int32),
        ],
        compiler_params=pltpu.CompilerParams(
            dimension_semantics=("parallel",),
        ),
    )(post, gm)

    x_hat, f = pl.pallas_call(
        _dec_kernel,
        grid=(nfb,),
        in_specs=[
            pl.BlockSpec((batch, bf), lambda i: (0, i)),
            pl.BlockSpec((d_model, bf), lambda i: (0, i)),
            pl.BlockSpec((d_model,), lambda i: (0,)),
            pl.BlockSpec((batch, 128), lambda i: (0, 0)),
        ],
        out_specs=[
            pl.BlockSpec((batch, d_model), lambda i: (0, 0)),
            pl.BlockSpec((batch, bf), lambda i: (0, i)),
        ],
        out_shape=[
            jax.ShapeDtypeStruct((batch, d_model), jnp.float32),
            jax.ShapeDtypeStruct((batch, n_features), jnp.float32),
        ],
        compiler_params=pltpu.CompilerParams(
            dimension_semantics=("arbitrary",),
        ),
    )(post, W_dec, b_dec, tf)

    return (x_hat, f)


# speedup vs baseline: 1.1365x; 1.1365x over previous
import functools

import jax
import jax.numpy as jnp
from jax.experimental import pallas as pl
from jax.experimental.pallas import tpu as pltpu

K = 64


def _enc_kernel(x_ref, w_ref, b_ref, post_ref, gm_ref):
    pre = jax.lax.dot_general(
        x_ref[...], w_ref[...],
        dimension_numbers=(((1,), (1,)), ((), ())),
        preferred_element_type=jnp.float32,
    )
    post = jnp.maximum(pre + b_ref[...][None, :], 0.0)
    post_ref[...] = post

    bits = jax.lax.bitcast_convert_type(post, jnp.int32)
    pm = bits[:, 0:128]
    for j in range(1, bits.shape[1] // 128):
        pm = jnp.maximum(pm, bits[:, j * 128:(j + 1) * 128])

    @pl.when(pl.program_id(0) == 0)
    def _():
        gm_ref[...] = pm

    @pl.when(pl.program_id(0) != 0)
    def _():
        gm_ref[...] = jnp.maximum(gm_ref[...], pm)


def _rowcount_ge(bits, mid, splits=8):
    n = bits.shape[1]
    w = n // splits
    tot = None
    for s in range(splits):
        c = jnp.sum((bits[:, s * w:(s + 1) * w] >= mid).astype(jnp.int32),
                    axis=1, keepdims=True)
        tot = c if tot is None else tot + c
    return tot


def _select_kernel(post_ref, gm_ref, t_ref,
                   lo_ref, hi_ref, clo_ref, chi_ref, tb_ref, *, k):
    post = post_ref[...]
    rows = post.shape[0]
    bits = jax.lax.bitcast_convert_type(post, jnp.int32)
    cm = gm_ref[...]

    def lb_body(i, p):
        cand = jnp.bitwise_or(p, jnp.int32(1) << (30 - i))
        cnt = jnp.sum((cm >= cand).astype(jnp.int32), axis=1, keepdims=True)
        return jnp.where(cnt >= k, cand, p)

    lo0 = jax.lax.fori_loop(0, 31, lb_body, jnp.zeros((rows, 1), jnp.int32))
    hi0 = jnp.max(cm, axis=1, keepdims=True) + 1

    cnt0 = _rowcount_ge(bits, lo0)
    lo_ref[...] = lo0
    hi_ref[...] = hi0
    clo_ref[...] = cnt0
    chi_ref[...] = jnp.zeros((rows, 1), jnp.int32)
    tb_ref[...] = jnp.where(cnt0 == k, lo0, jnp.full((rows, 1), -1, jnp.int32))

    def cond(state):
        it, n_act = state
        return jnp.logical_and(it < 60, n_act > 0)

    def step(it_s, use_interp):
        lo = lo_ref[...]
        hi = hi_ref[...]
        clo = clo_ref[...]
        chi = chi_ref[...]
        tb = tb_ref[...]
        active = (hi - lo > 1) & (tb < 0)

        bisect = lo + ((hi - lo) >> 1)
        if use_interp:
            lo_f = jax.lax.bitcast_convert_type(lo, jnp.float32)
            hi_f = jax.lax.bitcast_convert_type(hi, jnp.float32)
            frac = (clo - k).astype(jnp.float32) / jnp.maximum(
                (clo - chi).astype(jnp.float32), 1.0)
            interp = jax.lax.bitcast_convert_type(
                lo_f + frac * (hi_f - lo_f), jnp.int32)
            mid = jnp.where(it_s < 36, interp, bisect)
        else:
            mid = bisect
        mid = jnp.clip(mid, lo + 1, hi - 1)

        cnt = _rowcount_ge(bits, mid)
        hit = (cnt == k) & active
        ge = cnt >= k
        tb_ref[...] = jnp.where(hit, mid, tb)
        adv = active & ge & (~hit)
        lo_ref[...] = jnp.where(adv, mid, lo)
        clo_ref[...] = jnp.where(adv, cnt, clo)
        shr = active & (~ge)
        hi_ref[...] = jnp.where(shr, mid, hi)
        chi_ref[...] = jnp.where(shr, cnt, chi)

    def body(state):
        it, _ = state
        step(it, True)
        step(it + 1, True)
        step(it + 2, False)
        n_act = jnp.sum(((hi_ref[...] - lo_ref[...] > 1)
                         & (tb_ref[...] < 0)).astype(jnp.int32))
        return it + 3, n_act

    n0 = jnp.sum(((hi0 - lo0 > 1) & (cnt0 != k)).astype(jnp.int32))
    jax.lax.while_loop(cond, body, (jnp.int32(0), n0))
    thresh = jnp.where(tb_ref[...] < 0, lo_ref[...], tb_ref[...])
    t_ref[...] = jnp.broadcast_to(
        jax.lax.bitcast_convert_type(thresh, jnp.float32), t_ref.shape)


def _dec_kernel(post_ref, w_ref, b_ref, t_ref, xhat_ref, f_ref):
    post = post_ref[...]
    fblk = jnp.where(post >= t_ref[:, 0:1], post, 0.0)
    f_ref[...] = fblk

    @pl.when(pl.program_id(0) == 0)
    def _():
        xhat_ref[...] = jnp.broadcast_to(b_ref[...][None, :], xhat_ref.shape)

    xhat_ref[...] += jax.lax.dot_general(
        fblk, w_ref[...],
        dimension_numbers=(((1,), (1,)), ((), ())),
        preferred_element_type=jnp.float32,
    )


def kernel(x, W_enc, b_enc, W_dec, b_dec):
    batch, d_model = x.shape
    n_features = W_enc.shape[0]

    bf = min(1024, n_features)
    nfb = n_features // bf

    post, gm = pl.pallas_call(
        _enc_kernel,
        grid=(nfb,),
        in_specs=[
            pl.BlockSpec((batch, d_model), lambda i: (0, 0)),
            pl.BlockSpec((bf, d_model), lambda i: (i, 0)),
            pl.BlockSpec((bf,), lambda i: (i,)),
        ],
        out_specs=[
            pl.BlockSpec((batch, bf), lambda i: (0, i)),
            pl.BlockSpec((batch, 128), lambda i: (0, 0)),
        ],
        out_shape=[
            jax.ShapeDtypeStruct((batch, n_features), jnp.float32),
            jax.ShapeDtypeStruct((batch, 128), jnp.int32),
        ],
        compiler_params=pltpu.CompilerParams(
            dimension_semantics=("arbitrary",),
        ),
    )(x, W_enc, b_enc)

    rb = min(128, batch)
    tf = pl.pallas_call(
        functools.partial(_select_kernel, k=K),
        grid=(batch // rb,),
        in_specs=[
            pl.BlockSpec((rb, n_features), lambda i: (i, 0)),
            pl.BlockSpec((rb, 128), lambda i: (i, 0)),
        ],
        out_specs=pl.BlockSpec((rb, 128), lambda i: (i, 0)),
        out_shape=jax.ShapeDtypeStruct((batch, 128), jnp.float32),
        scratch_shapes=[
            pltpu.VMEM((rb, 1), jnp.int32),
            pltpu.VMEM((rb, 1), jnp.int32),
            pltpu.VMEM((rb, 1), jnp.int32),
            pltpu.VMEM((rb, 1), jnp.int32),
            pltpu.VMEM((rb, 1), jnp.int32),
        ],
        compiler_params=pltpu.CompilerParams(
            dimension_semantics=("parallel",),
        ),
    )(post, gm)

    x_hat, f = pl.pallas_call(
        _dec_kernel,
        grid=(nfb,),
        in_specs=[
            pl.BlockSpec((batch, bf), lambda i: (0, i)),
            pl.BlockSpec((d_model, bf), lambda i: (0, i)),
            pl.BlockSpec((d_model,), lambda i: (0,)),
            pl.BlockSpec((batch, 128), lambda i: (0, 0)),
        ],
        out_specs=[
            pl.BlockSpec((batch, d_model), lambda i: (0, 0)),
            pl.BlockSpec((batch, bf), lambda i: (0, i)),
        ],
        out_shape=[
            jax.ShapeDtypeStruct((batch, d_model), jnp.float32),
            jax.ShapeDtypeStruct((batch, n_features), jnp.float32),
        ],
        compiler_params=pltpu.CompilerParams(
            dimension_semantics=("arbitrary",),
        ),
    )(post, W_dec, b_dec, tf)

    return (x_hat, f)


# speedup vs baseline: 1.1790x; 1.0374x over previous
import functools

import jax
import jax.numpy as jnp
from jax.experimental import pallas as pl
from jax.experimental.pallas import tpu as pltpu

K = 64


def _enc_kernel(x_ref, w_ref, b_ref, post_ref, gm_ref):
    pre = jax.lax.dot_general(
        x_ref[...], w_ref[...],
        dimension_numbers=(((1,), (1,)), ((), ())),
        preferred_element_type=jnp.float32,
    )
    post = jnp.maximum(pre + b_ref[...][None, :], 0.0)
    post_ref[...] = post

    bits = jax.lax.bitcast_convert_type(post, jnp.int32)
    pm = bits[:, 0:128]
    for j in range(1, bits.shape[1] // 128):
        pm = jnp.maximum(pm, bits[:, j * 128:(j + 1) * 128])

    @pl.when(pl.program_id(0) == 0)
    def _():
        gm_ref[...] = pm

    @pl.when(pl.program_id(0) != 0)
    def _():
        gm_ref[...] = jnp.maximum(gm_ref[...], pm)


def _rowcount_ge(bits, mid, splits=8):
    n = bits.shape[1]
    w = n // splits
    tot = None
    for s in range(splits):
        c = jnp.sum((bits[:, s * w:(s + 1) * w] >= mid).astype(jnp.int32),
                    axis=1, keepdims=True)
        tot = c if tot is None else tot + c
    return tot


def _select_kernel(post_ref, gm_ref, t_ref,
                   lo_ref, hi_ref, clo_ref, chi_ref, tb_ref, *, k):
    post = post_ref[...]
    rows = post.shape[0]
    bits = jax.lax.bitcast_convert_type(post, jnp.int32)
    cm = gm_ref[...]

    def lb_body(i, p):
        cand = jnp.bitwise_or(p, jnp.int32(1) << (30 - i))
        cnt = jnp.sum((cm >= cand).astype(jnp.int32), axis=1, keepdims=True)
        return jnp.where(cnt >= k, cand, p)

    lo0 = jax.lax.fori_loop(0, 31, lb_body, jnp.zeros((rows, 1), jnp.int32))
    hi0 = jnp.max(cm, axis=1, keepdims=True) + 1

    cnt0 = _rowcount_ge(bits, lo0)
    lo_ref[...] = lo0
    hi_ref[...] = hi0
    clo_ref[...] = cnt0
    chi_ref[...] = jnp.zeros((rows, 1), jnp.int32)
    tb_ref[...] = jnp.where(cnt0 == k, lo0, jnp.full((rows, 1), -1, jnp.int32))

    def cond(state):
        it, n_act = state
        return jnp.logical_and(it < 60, n_act > 0)

    def step(it_s, use_interp):
        lo = lo_ref[...]
        hi = hi_ref[...]
        clo = clo_ref[...]
        chi = chi_ref[...]
        tb = tb_ref[...]
        active = (hi - lo > 1) & (tb < 0)

        bisect = lo + ((hi - lo) >> 1)
        if use_interp:
            lo_f = jax.lax.bitcast_convert_type(lo, jnp.float32)
            hi_f = jax.lax.bitcast_convert_type(hi, jnp.float32)
            frac = (clo - k).astype(jnp.float32) / jnp.maximum(
                (clo - chi).astype(jnp.float32), 1.0)
            interp = jax.lax.bitcast_convert_type(
                lo_f + frac * (hi_f - lo_f), jnp.int32)
            mid = jnp.where(it_s < 36, interp, bisect)
        else:
            mid = bisect
        mid = jnp.clip(mid, lo + 1, hi - 1)

        cnt = _rowcount_ge(bits, mid)
        hit = (cnt == k) & active
        ge = cnt >= k
        tb_ref[...] = jnp.where(hit, mid, tb)
        adv = active & ge & (~hit)
        lo_ref[...] = jnp.where(adv, mid, lo)
        clo_ref[...] = jnp.where(adv, cnt, clo)
        shr = active & (~ge)
        hi_ref[...] = jnp.where(shr, mid, hi)
        chi_ref[...] = jnp.where(shr, cnt, chi)

    def body(state):
        it, _ = state
        step(it, True)
        step(it + 1, True)
        step(it + 2, False)
        n_act = jnp.sum(((hi_ref[...] - lo_ref[...] > 1)
                         & (tb_ref[...] < 0)).astype(jnp.int32))
        return it + 3, n_act

    n0 = jnp.sum(((hi0 - lo0 > 1) & (cnt0 != k)).astype(jnp.int32))
    jax.lax.while_loop(cond, body, (jnp.int32(0), n0))
    thresh = jnp.where(tb_ref[...] < 0, lo_ref[...], tb_ref[...])
    t_ref[...] = jnp.broadcast_to(
        jax.lax.bitcast_convert_type(thresh, jnp.float32), t_ref.shape)


def _dec_kernel(post_ref, w_ref, b_ref, t_ref, xhat_ref, f_ref):
    post = post_ref[...]
    fblk = jnp.where(post >= t_ref[:, 0:1], post, 0.0)
    f_ref[...] = fblk

    @pl.when(pl.program_id(0) == 0)
    def _():
        xhat_ref[...] = jnp.broadcast_to(b_ref[...][None, :], xhat_ref.shape)

    xhat_ref[...] += jax.lax.dot_general(
        fblk, w_ref[...],
        dimension_numbers=(((1,), (1,)), ((), ())),
        preferred_element_type=jnp.float32,
    )


def kernel(x, W_enc, b_enc, W_dec, b_dec):
    batch, d_model = x.shape
    n_features = W_enc.shape[0]

    bf = min(1024, n_features)
    nfb = n_features // bf

    post, gm = pl.pallas_call(
        _enc_kernel,
        grid=(nfb,),
        in_specs=[
            pl.BlockSpec((batch, d_model), lambda i: (0, 0)),
            pl.BlockSpec((bf, d_model), lambda i: (i, 0)),
            pl.BlockSpec((bf,), lambda i: (i,)),
        ],
        out_specs=[
            pl.BlockSpec((batch, bf), lambda i: (0, i)),
            pl.BlockSpec((batch, 128), lambda i: (0, 0)),
        ],
        out_shape=[
            jax.ShapeDtypeStruct((batch, n_features), jnp.float32),
            jax.ShapeDtypeStruct((batch, 128), jnp.int32),
        ],
        compiler_params=pltpu.CompilerParams(
            dimension_semantics=("arbitrary",),
        ),
    )(x, W_enc, b_enc)

    rb = min(256, batch)
    tf = pl.pallas_call(
        functools.partial(_select_kernel, k=K),
        grid=(batch // rb,),
        in_specs=[
            pl.BlockSpec((rb, n_features), lambda i: (i, 0)),
            pl.BlockSpec((rb, 128), lambda i: (i, 0)),
        ],
        out_specs=pl.BlockSpec((rb, 128), lambda i: (i, 0)),
        out_shape=jax.ShapeDtypeStruct((batch, 128), jnp.float32),
        scratch_shapes=[
            pltpu.VMEM((rb, 1), jnp.int32),
            pltpu.VMEM((rb, 1), jnp.int32),
            pltpu.VMEM((rb, 1), jnp.int32),
            pltpu.VMEM((rb, 1), jnp.int32),
            pltpu.VMEM((rb, 1), jnp.int32),
        ],
        compiler_params=pltpu.CompilerParams(
            dimension_semantics=("parallel",),
        ),
    )(post, gm)

    x_hat, f = pl.pallas_call(
        _dec_kernel,
        grid=(nfb,),
        in_specs=[
            pl.BlockSpec((batch, bf), lambda i: (0, i)),
            pl.BlockSpec((d_model, bf), lambda i: (0, i)),
            pl.BlockSpec((d_model,), lambda i: (0,)),
            pl.BlockSpec((batch, 128), lambda i: (0, 0)),
        ],
        out_specs=[
            pl.BlockSpec((batch, d_model), lambda i: (0, 0)),
            pl.BlockSpec((batch, bf), lambda i: (0, i)),
        ],
        out_shape=[
            jax.ShapeDtypeStruct((batch, d_model), jnp.float32),
            jax.ShapeDtypeStruct((batch, n_features), jnp.float32),
        ],
        compiler_params=pltpu.CompilerParams(
            dimension_semantics=("arbitrary",),
        ),
    )(post, W_dec, b_dec, tf)

    return (x_hat, f)
